# Initial kernel scaffold; baseline (speedup 1.0000x reference)
import jax
import jax.numpy as jnp
from jax.experimental import pallas as pl


def kernel(x, table):
    raise NotImplementedError("write your pallas kernel here")


# speedup vs baseline: 1.4382x; 1.4382x over previous
import jax
import jax.numpy as jnp
from jax import lax
from jax.experimental import pallas as pl
from jax.experimental.pallas import tpu as pltpu
from jax.experimental.pallas import tpu_sc as plsc

_KEEP = 26
_EMB = 32
_BATCH = 16384
_L = 128
_NW = 32
_FLAT = _BATCH * _KEEP
_NROWS = _FLAT // _L
_CHUNKS = _NROWS // _NW


def _gather_body(idx_hbm, table_hbm, out_hbm, idx_v, rows_v, gsem):
    wid = lax.axis_index("s") * 2 + lax.axis_index("c")
    base = wid * _CHUNKS
    pltpu.sync_copy(idx_hbm.at[pl.ds(base, _CHUNKS)], idx_v)

    def body(j, carry):
        pltpu.async_copy(table_hbm.at[idx_v.at[j]], rows_v, gsem).wait()
        pltpu.sync_copy(rows_v, out_hbm.at[pl.ds((base + j) * _L, _L)])
        return carry

    lax.fori_loop(0, _CHUNKS, body, 0)


def kernel(x, table):
    idx = x[:, :_KEEP].reshape(_NROWS, _L)
    mesh = plsc.VectorSubcoreMesh(core_axis_name="c", subcore_axis_name="s")
    out = pl.kernel(
        _gather_body,
        mesh=mesh,
        compiler_params=pltpu.CompilerParams(use_tc_tiling_on_sc=False),
        out_type=jax.ShapeDtypeStruct((_FLAT, _EMB), jnp.float32),
        scratch_types=[
            pltpu.VMEM((_CHUNKS, _L), jnp.int32),
            pltpu.VMEM((_L, _EMB), jnp.float32),
            pltpu.SemaphoreType.DMA,
        ],
    )(idx, table)
    return out.reshape(_BATCH, _KEEP, _EMB)


# speedup vs baseline: 1.5705x; 1.0920x over previous
import jax
import jax.numpy as jnp
from jax import lax
from jax.experimental import pallas as pl
from jax.experimental.pallas import tpu as pltpu
from jax.experimental.pallas import tpu_sc as plsc

_KEEP = 26
_EMB = 32
_BATCH = 16384
_L = 128
_NW = 32
_FLAT = _BATCH * _KEEP
_NROWS = _FLAT // _L
_CHUNKS = _NROWS // _NW
_BLK = 13
_NBLK = _CHUNKS // _BLK
_BROWS = _BLK * _L


def _gather_body(idx_hbm, table_hbm, out_hbm, idx_v, buf0, buf1, g0, g1, s0, s1):
    wid = lax.axis_index("s") * 2 + lax.axis_index("c")
    base = wid * _CHUNKS
    pltpu.sync_copy(idx_hbm.at[pl.ds(base, _CHUNKS)], idx_v)

    def issue(blk, buf, gsem):
        for b in range(_BLK):
            pltpu.async_copy(
                table_hbm.at[idx_v.at[blk * _BLK + b]],
                buf.at[pl.ds(b * _L, _L)],
                gsem,
            )

    def wait_gathers(buf, gsem):
        pltpu.make_async_copy(table_hbm.at[pl.ds(0, _BROWS)], buf, gsem).wait()

    def store(blk, buf, ssem):
        pltpu.async_copy(
            buf, out_hbm.at[pl.ds((base + blk * _BLK) * _L, _BROWS)], ssem
        )

    def wait_store(blk, buf, ssem):
        pltpu.make_async_copy(
            buf, out_hbm.at[pl.ds((base + blk * _BLK) * _L, _BROWS)], ssem
        ).wait()

    issue(0, buf0, g0)
    issue(1, buf1, g1)

    def body(i, carry):
        blk0 = 2 * i
        wait_gathers(buf0, g0)
        store(blk0, buf0, s0)
        wait_gathers(buf1, g1)
        store(blk0 + 1, buf1, s1)
        wait_store(blk0, buf0, s0)
        issue(blk0 + 2, buf0, g0)
        wait_store(blk0 + 1, buf1, s1)
        issue(blk0 + 3, buf1, g1)
        return carry

    lax.fori_loop(0, _NBLK // 2 - 1, body, 0)

    last = _NBLK - 2
    wait_gathers(buf0, g0)
    store(last, buf0, s0)
    wait_gathers(buf1, g1)
    store(last + 1, buf1, s1)
    wait_store(last, buf0, s0)
    wait_store(last + 1, buf1, s1)


def kernel(x, table):
    idx = x[:, :_KEEP].reshape(_NROWS, _L)
    mesh = plsc.VectorSubcoreMesh(core_axis_name="c", subcore_axis_name="s")
    out = pl.kernel(
        _gather_body,
        mesh=mesh,
        compiler_params=pltpu.CompilerParams(use_tc_tiling_on_sc=False),
        out_type=jax.ShapeDtypeStruct((_FLAT, _EMB), jnp.float32),
        scratch_types=[
            pltpu.VMEM((_CHUNKS, _L), jnp.int32),
            pltpu.VMEM((_BROWS, _EMB), jnp.float32),
            pltpu.VMEM((_BROWS, _EMB), jnp.float32),
            pltpu.SemaphoreType.DMA,
            pltpu.SemaphoreType.DMA,
            pltpu.SemaphoreType.DMA,
            pltpu.SemaphoreType.DMA,
        ],
    )(idx, table)
    return out.reshape(_BATCH, _KEEP, _EMB)
